# baseline (speedup 1.0000x reference)
import functools

import jax
import jax.numpy as jnp
from jax import lax
from jax.experimental import pallas as pl
from jax.experimental.pallas import tpu as pltpu
from jax.experimental.pallas import tpu_sc as plsc

_L = 16
_NS = 16
_NROWS = _NS * _L
_D = 64


def _sc_body(emb, idx, out, idx_v, sidx_v, rows_v, srows_v, part_v, comb_v,
             out_v, shared, sem):
    w = lax.axis_index("s")
    base = pl.multiple_of(w * _L, _L)

    pltpu.sync_copy(idx.at[pl.ds(base, _L)], idx_v)
    pltpu.sync_copy(idx.at[pl.ds(_NROWS, 8)], sidx_v)

    cp_rows = pltpu.async_copy(emb.at[idx_v], rows_v, sem)
    cp_src = pltpu.async_copy(emb.at[sidx_v], srows_v, sem)
    cp_rows.wait()
    cp_src.wait()

    lanes = lax.iota(jnp.int32, _L)
    acc = jnp.zeros((_L,), jnp.float32)
    src_chunks = [srows_v[0, pl.ds(c * _L, _L)] for c in range(_D // _L)]
    for d in range(_D):
        col = plsc.load_gather(rows_v, [lanes, jnp.full((_L,), d, jnp.int32)])
        acc = acc + col * src_chunks[d // _L][d % _L]

    lane_r = lanes + base
    neg_mask = lane_r < 200
    ctx_mask = jnp.logical_and(lane_r >= 200, lane_r < 250)
    sig = 1.0 / (1.0 + jnp.exp(acc))
    part_v[0, :] = jnp.where(neg_mask, sig, 0.0)
    part_v[1, :] = jnp.where(ctx_mask, acc, 0.0)
    pltpu.sync_copy(part_v, shared.at[pl.ds(2 * w, 2)])
    plsc.subcore_barrier()

    @pl.when(w == 0)
    def _():
        pltpu.sync_copy(shared, comb_v)
        nacc = jnp.zeros((_L,), jnp.float32)
        pacc = jnp.zeros((_L,), jnp.float32)
        for i in range(_NS):
            nacc = nacc + comb_v[2 * i, :]
            pacc = pacc + comb_v[2 * i + 1, :]
        nsum = jnp.sum(nacc)
        psum = jnp.sum(pacc)
        pos = 1.0 / (1.0 + jnp.exp(-(jnp.zeros((_L,), jnp.float32) + psum)))
        posc = jnp.clip(pos, 1e-7, 1.0 - 1e-7)
        negc = jnp.clip(jnp.zeros((_L,), jnp.float32) + nsum, 1e-7, 1.0 - 1e-7)
        out_v[...] = jnp.where(lanes == 0, posc, negc)
        pltpu.sync_copy(out_v, out)


@functools.partial(jax.jit, static_argnums=())
def _sc_loss_parts(embedding, idx):
    f = pl.kernel(
        _sc_body,
        out_type=jax.ShapeDtypeStruct((_L,), jnp.float32),
        mesh=plsc.VectorSubcoreMesh(
            core_axis_name="c", subcore_axis_name="s",
            num_cores=1, num_subcores=_NS),
        scratch_types=[
            pltpu.VMEM((_L,), jnp.int32),
            pltpu.VMEM((8,), jnp.int32),
            pltpu.VMEM((_L, _D), jnp.float32),
            pltpu.VMEM((8, _D), jnp.float32),
            pltpu.VMEM((2, _L), jnp.float32),
            pltpu.VMEM((2 * _NS, _L), jnp.float32),
            pltpu.VMEM((_L,), jnp.float32),
            pltpu.VMEM_SHARED((2 * _NS, _L), jnp.float32),
            pltpu.SemaphoreType.DMA,
        ],
        compiler_params=pltpu.CompilerParams(
            needs_layout_passes=False, use_tc_tiling_on_sc=False),
    )
    return f(embedding, idx)


def kernel(embedding, source_node, context_nodes, neg_samples):
    idx = jnp.concatenate([
        neg_samples.astype(jnp.int32),
        context_nodes.astype(jnp.int32),
        jnp.zeros((6,), jnp.int32),
        jnp.broadcast_to(source_node.astype(jnp.int32), (8,)),
    ])
    parts = _sc_loss_parts(embedding, idx)
    return -jnp.log(parts[0]) - parts[1]
